# baseline (speedup 1.0000x reference)
import functools

import jax
import jax.numpy as jnp
from jax import lax
from jax.experimental import pallas as pl
from jax.experimental.pallas import tpu as pltpu
from jax.experimental.pallas import tpu_sc as plsc

N_CELLS = 16384
N_HIDDEN = 256
N_LABELS = 512
NC = 2
NS = 16
FH = 128
CHUNK = 128
CPT = 16
CELLS_PER_TILE = CHUNK * CPT


def _sc_body(pl_lat, pl_lab, tr_lat, tr_lab, parts, cnts,
             table, rowbuf, labbuf, cnt):
    i32 = jnp.int32
    f32 = jnp.float32
    cid = lax.axis_index("c")
    sid = lax.axis_index("s")
    half = sid % 2
    grp = sid // 2
    wid = cid * NS + sid

    zeros16 = jnp.zeros((16,), f32)
    iota16 = lax.iota(i32, 16)
    ones16 = jnp.ones((16,), f32)
    mask0 = iota16 < 1
    zeros16_i = jnp.zeros((16,), i32)

    def ztab(r, c):
        for k in range(N_LABELS // 16):
            table[r, pl.ds(k * 16, 16)] = zeros16
        return c
    lax.fori_loop(0, FH, ztab, 0)

    def zcnt(r, c):
        cnt[0, pl.ds(r * 16, 16)] = zeros16
        return c
    lax.fori_loop(0, N_LABELS // 16, zcnt, 0)

    def run(lat, lab):
        pltpu.sync_copy(lab.at[pl.ds(grp * CELLS_PER_TILE, CELLS_PER_TILE)],
                        labbuf)
        for j in range(CPT):
            base = grp * CELLS_PER_TILE + j * CHUNK
            pltpu.sync_copy(lat.at[pl.ds(base, CHUNK), pl.ds(half * FH, FH)],
                            rowbuf)

            def cell(ci, c):
                splat_cell = jnp.full((16,), j * CHUNK, i32) + ci
                lab_splat = plsc.load_gather(labbuf, [splat_cell])
                splat_ci = jnp.full((16,), ci, i32)
                for k in range(FH // 16):
                    row_k = plsc.load_gather(rowbuf, [splat_ci, k * 16 + iota16])
                    plsc.addupdate_scatter(table, [k * 16 + iota16, lab_splat],
                                           row_k)
                plsc.addupdate_scatter(cnt, [zeros16_i, lab_splat], ones16,
                                       mask=mask0)
                return c
            lax.fori_loop(0, CHUNK, cell, 0)

        pltpu.sync_copy(table, parts.at[wid])
        pltpu.sync_copy(cnt, cnts.at[wid])

    @pl.when(cid == 0)
    def _pseudo():
        run(pl_lat, pl_lab)

    @pl.when(cid == 1)
    def _true():
        run(tr_lat, tr_lab)


def _combine_kernel(parts, cnts, cp, ct, pc, tc, nc, out):
    def assemble(base):
        low = parts[base]
        high = parts[base + 1]
        for g in range(1, NS // 2):
            low = low + parts[base + 2 * g]
            high = high + parts[base + 2 * g + 1]
        return jnp.concatenate([low, high], axis=0)

    def count_sum(base):
        c = cnts[base]
        for j in range(1, NS):
            c = c + cnts[base + j]
        return c * 0.5

    sums_p = assemble(0)
    sums_t = assemble(NS)
    counts_p = count_sum(0)
    counts_t = count_sum(NS)

    reset = jnp.max(pc[...]) >= float(N_LABELS) * 1000.0
    pcs = jnp.where(reset, jnp.ones_like(pc[...]), pc[...])
    tcs = tc[...]

    cent_p = sums_p / jnp.maximum(counts_p, 1.0)
    mask_p = counts_p > 5.0
    cp_new = jnp.where(mask_p,
                       (cp[...] * pcs + cent_p * counts_p) / (pcs + counts_p),
                       cp[...])

    cent_t = sums_t / jnp.maximum(counts_t, 1.0)
    mask_t = counts_t > 5.0
    ct_new = jnp.where(mask_t,
                       (ct[...] * tcs + cent_t * counts_t) / (tcs + counts_t),
                       ct[...])

    mse = jnp.sum((cp_new - ct_new) ** 2) / float(N_LABELS * N_HIDDEN)

    pc_new = jnp.where(mask_p, pcs + counts_p, pcs)
    t = pc_new / jnp.sum(pc_new)
    kl = jnp.sum(jnp.where(t > 0.0,
                           t * (jnp.log(t) - jnp.log(nc[...])),
                           0.0)) / float(N_LABELS)
    out[...] = jnp.reshape(mse + kl, (1, 1))


@jax.jit
def _run(pl_lat, pl_lab, tr_lat, tr_lab, cp, ct, pc, tc, nc):
    f32 = jnp.float32
    sc_segsum = functools.partial(
        pl.kernel,
        out_type=(
            jax.ShapeDtypeStruct((NC * NS, FH, N_LABELS), f32),
            jax.ShapeDtypeStruct((NC * NS, 1, N_LABELS), f32),
        ),
        mesh=plsc.VectorSubcoreMesh(core_axis_name="c", subcore_axis_name="s"),
        compiler_params=pltpu.CompilerParams(needs_layout_passes=False),
        scratch_types=[
            pltpu.VMEM((FH, N_LABELS), f32),
            pltpu.VMEM((CHUNK, FH), f32),
            pltpu.VMEM((CELLS_PER_TILE,), jnp.int32),
            pltpu.VMEM((1, N_LABELS), f32),
        ],
    )(_sc_body)
    parts, cnts = sc_segsum(pl_lat, pl_lab, tr_lat, tr_lab)

    out = pl.pallas_call(
        _combine_kernel,
        out_shape=jax.ShapeDtypeStruct((1, 1), f32),
    )(parts, cnts, cp, ct, pc, tc, nc)
    return out[0, 0]


def kernel(pseudo_latent, pseudo_labels, true_latent, true_labels,
           centroids_pseudo, pseudo_count, centroids_true, true_count, ncells):
    pl_lab = pseudo_labels.astype(jnp.int32)
    tr_lab = true_labels.astype(jnp.int32)
    pc = pseudo_count.reshape(1, N_LABELS)
    tc = true_count.reshape(1, N_LABELS)
    nc = ncells.reshape(1, N_LABELS)
    return _run(pseudo_latent, pl_lab, true_latent, tr_lab,
                centroids_pseudo, centroids_true, pc, tc, nc)


# speedup vs baseline: 4.7650x; 4.7650x over previous
import functools

import jax
import jax.numpy as jnp
from jax import lax
from jax.experimental import pallas as pl
from jax.experimental.pallas import tpu as pltpu
from jax.experimental.pallas import tpu_sc as plsc

N_CELLS = 16384
N_HIDDEN = 256
N_LABELS = 512
NC = 2
NS = 16
FH = 128
CHUNK = 128
CPT = 16
CELLS_PER_TILE = CHUNK * CPT


def _sc_body(pl_lat, pl_lab, tr_lat, tr_lab, parts, cnts,
             table, rowbuf0, rowbuf1, labbuf, cnt, sem0, sem1):
    i32 = jnp.int32
    f32 = jnp.float32
    cid = lax.axis_index("c")
    sid = lax.axis_index("s")
    half = sid % 2
    grp = sid // 2
    wid = cid * NS + sid

    zeros16 = jnp.zeros((16,), f32)
    iota16 = lax.iota(i32, 16)
    ones16 = jnp.ones((16,), f32)
    mask0 = iota16 < 1
    zeros16_i = jnp.zeros((16,), i32)

    def _ztab(r, c):
        for k in range(FH // 16):
            table[r, pl.ds(k * 16, 16)] = zeros16
        return c
    lax.fori_loop(0, N_LABELS, _ztab, 0)

    def _zcnt(r, c):
        cnt[0, pl.ds(r * 16, 16)] = zeros16
        return c
    lax.fori_loop(0, N_LABELS // 16, _zcnt, 0)

    bufs = (rowbuf0, rowbuf1)
    sems = (sem0, sem1)

    def run(lat, lab):
        pltpu.sync_copy(lab.at[pl.ds(grp * CELLS_PER_TILE, CELLS_PER_TILE)],
                        labbuf)

        def chunk_slice(j):
            base = grp * CELLS_PER_TILE + j * CHUNK
            return lat.at[pl.ds(base, CHUNK), pl.ds(half * FH, FH)]

        pltpu.async_copy(chunk_slice(0), bufs[0], sems[0])

        def pair(jj, c):
            for phase in range(2):
                j = jj * 2 + phase
                nxt = j + 1

                @pl.when(nxt < CPT)
                def _():
                    pltpu.async_copy(chunk_slice(nxt), bufs[(phase + 1) % 2],
                                     sems[(phase + 1) % 2])

                pltpu.make_async_copy(chunk_slice(j), bufs[phase],
                                      sems[phase]).wait()
                buf = bufs[phase]

                @plsc.parallel_loop(0, CHUNK, unroll=4)
                def _cell(ci):
                    splat_cell = jnp.full((16,), j * CHUNK, i32) + ci
                    lab_splat = plsc.load_gather(labbuf, [splat_cell])
                    for k in range(FH // 16):
                        row_k = buf[ci, pl.ds(k * 16, 16)]
                        plsc.addupdate_scatter(
                            table, [lab_splat, k * 16 + iota16], row_k)
                    plsc.addupdate_scatter(cnt, [zeros16_i, lab_splat], ones16,
                                           mask=mask0)
            return c
        lax.fori_loop(0, CPT // 2, pair, 0)

        pltpu.sync_copy(table, parts.at[wid])
        pltpu.sync_copy(cnt, cnts.at[wid])

    @pl.when(cid == 0)
    def _pseudo():
        run(pl_lat, pl_lab)

    @pl.when(cid == 1)
    def _true():
        run(tr_lat, tr_lab)


def _combine_kernel(parts, cnts, cp_f, ct_f, pc, tc, nc, out):
    cp = jnp.transpose(cp_f[...], (1, 0))
    ct = jnp.transpose(ct_f[...], (1, 0))
    def assemble(base):
        low = parts[base]
        high = parts[base + 1]
        for g in range(1, NS // 2):
            low = low + parts[base + 2 * g]
            high = high + parts[base + 2 * g + 1]
        return jnp.concatenate([low, high], axis=1)

    def count_sum(base):
        c = cnts[base]
        for j in range(1, NS):
            c = c + cnts[base + j]
        return jnp.reshape(c * 0.5, (N_LABELS, 1))

    sums_p = assemble(0)
    sums_t = assemble(NS)
    counts_p = count_sum(0)
    counts_t = count_sum(NS)

    reset = jnp.max(pc[...]) >= float(N_LABELS) * 1000.0
    pcs = jnp.where(reset, jnp.ones_like(pc[...]), pc[...])
    tcs = tc[...]

    cent_p = sums_p / jnp.maximum(counts_p, 1.0)
    mask_p = counts_p > 5.0
    cp_new = jnp.where(mask_p,
                       (cp * pcs + cent_p * counts_p) / (pcs + counts_p),
                       cp)

    cent_t = sums_t / jnp.maximum(counts_t, 1.0)
    mask_t = counts_t > 5.0
    ct_new = jnp.where(mask_t,
                       (ct * tcs + cent_t * counts_t) / (tcs + counts_t),
                       ct)

    mse = jnp.sum((cp_new - ct_new) ** 2) / float(N_LABELS * N_HIDDEN)

    pc_new = jnp.where(mask_p, pcs + counts_p, pcs)
    t = pc_new / jnp.sum(pc_new)
    kl = jnp.sum(jnp.where(t > 0.0,
                           t * (jnp.log(t) - jnp.log(nc[...])),
                           0.0)) / float(N_LABELS)
    out[...] = jnp.reshape(mse + kl, (1, 1))


@jax.jit
def _run(pl_lat, pl_lab, tr_lat, tr_lab, cp, ct, pc, tc, nc):
    f32 = jnp.float32
    sc_segsum = functools.partial(
        pl.kernel,
        out_type=(
            jax.ShapeDtypeStruct((NC * NS, N_LABELS, FH), f32),
            jax.ShapeDtypeStruct((NC * NS, 1, N_LABELS), f32),
        ),
        mesh=plsc.VectorSubcoreMesh(core_axis_name="c", subcore_axis_name="s"),
        compiler_params=pltpu.CompilerParams(needs_layout_passes=False),
        scratch_types=[
            pltpu.VMEM((N_LABELS, FH), f32),
            pltpu.VMEM((CHUNK, FH), f32),
            pltpu.VMEM((CHUNK, FH), f32),
            pltpu.VMEM((CELLS_PER_TILE,), jnp.int32),
            pltpu.VMEM((1, N_LABELS), f32),
            pltpu.SemaphoreType.DMA,
            pltpu.SemaphoreType.DMA,
        ],
    )(_sc_body)
    parts, cnts = sc_segsum(pl_lat, pl_lab, tr_lat, tr_lab)

    out = pl.pallas_call(
        _combine_kernel,
        out_shape=jax.ShapeDtypeStruct((1, 1), f32),
    )(parts, cnts, cp, ct, pc, tc, nc)
    return out[0, 0]


def kernel(pseudo_latent, pseudo_labels, true_latent, true_labels,
           centroids_pseudo, pseudo_count, centroids_true, true_count, ncells):
    pl_lab = pseudo_labels.astype(jnp.int32)
    tr_lab = true_labels.astype(jnp.int32)
    pc = pseudo_count.reshape(N_LABELS, 1)
    tc = true_count.reshape(N_LABELS, 1)
    nc = ncells.reshape(N_LABELS, 1)
    return _run(pseudo_latent, pl_lab, true_latent, tr_lab,
                centroids_pseudo, centroids_true, pc, tc, nc)
